# baseline (speedup 1.0000x reference)
import functools

import jax
import jax.numpy as jnp
from jax import lax
from jax.experimental import pallas as pl
from jax.experimental.pallas import tpu as pltpu
from jax.experimental.pallas import tpu_sc as plsc

_NS = 16
_L = 16
_N = 65536
_NW = _NS
_CHUNK = _N // _NW
_NVEC = _CHUNK // _L
_NBIN = 16
_UNROLL = 4
_NBANK = 4
_HALF = _CHUNK // 2


def _sc_body(idx_hbm, vals_hbm, out_hbm,
             rc_v, val_v, acc_v, part_v, red_v, out_v, shared,
             sem_a, sem_b):
    s = lax.axis_index("s")
    base = s * _CHUNK
    cps_a = [
        pltpu.async_copy(idx_hbm.at[:, pl.ds(base, _HALF)],
                         rc_v.at[:, pl.ds(0, _HALF)], sem_a),
        pltpu.async_copy(vals_hbm.at[pl.ds(base, _HALF)],
                         val_v.at[pl.ds(0, _HALF)], sem_a),
    ]
    cps_b = [
        pltpu.async_copy(idx_hbm.at[:, pl.ds(base + _HALF, _HALF)],
                         rc_v.at[:, pl.ds(_HALF, _HALF)], sem_b),
        pltpu.async_copy(vals_hbm.at[pl.ds(base + _HALF, _HALF)],
                         val_v.at[pl.ds(_HALF, _HALF)], sem_b),
    ]

    zero = jnp.zeros((_L,), jnp.float32)
    for i in range(_NBANK * _NBIN):
        acc_v[pl.ds(i * _L, _L)] = zero

    lane16 = lax.iota(jnp.int32, _L) * _NBIN

    def _loop(i):
        off = i * _L
        r = rc_v[0, pl.ds(off, _L)]
        cc = rc_v[1, pl.ds(off, _L)]
        v = val_v[pl.ds(off, _L)]
        bank = (i % _NBANK) * (_NBIN * _L)
        idx = bank + lane16 + r * 4 + cc
        plsc.addupdate_scatter(acc_v, [idx], v)

    for cp in cps_a:
        cp.wait()
    plsc.parallel_loop(0, _NVEC // 2, 1, unroll=_UNROLL)(_loop)
    for cp in cps_b:
        cp.wait()
    plsc.parallel_loop(_NVEC // 2, _NVEC, 1, unroll=_UNROLL)(_loop)

    part = acc_v[pl.ds(0, _L)]
    for l in range(1, _NBANK * _NS):
        part = part + acc_v[pl.ds(l * _L, _L)]
    part_v[...] = part

    pltpu.sync_copy(part_v, shared.at[pl.ds(s * _L, _L)])
    plsc.subcore_barrier()

    @pl.when(s == 0)
    def _():
        pltpu.sync_copy(shared, red_v)
        tot = red_v[pl.ds(0, _L)]
        for l in range(1, _NS):
            tot = tot + red_v[pl.ds(l * _L, _L)]
        lane = lax.iota(jnp.int32, _L)
        plsc.store_scatter(out_v, [lane // 4, lane % 4], tot)
        pltpu.sync_copy(out_v, out_hbm)


_sc_scatter = functools.partial(
    pl.kernel,
    out_type=jax.ShapeDtypeStruct((4, 4), jnp.float32),
    mesh=plsc.VectorSubcoreMesh(
        core_axis_name="c", subcore_axis_name="s", num_cores=1),
    compiler_params=pltpu.CompilerParams(needs_layout_passes=False),
    scratch_types=[
        pltpu.VMEM((2, _CHUNK), jnp.int32),
        pltpu.VMEM((_CHUNK,), jnp.float32),
        pltpu.VMEM((_NBANK * _NBIN * _L,), jnp.float32),
        pltpu.VMEM((_L,), jnp.float32),
        pltpu.VMEM((_NS * _L,), jnp.float32),
        pltpu.VMEM((4, 4), jnp.float32),
        pltpu.VMEM_SHARED((_NS * _L,), jnp.float32),
        pltpu.SemaphoreType.DMA,
        pltpu.SemaphoreType.DMA,
    ],
)(_sc_body)


def kernel(indices, values):
    return _sc_scatter(indices.astype(jnp.int32), values)


# speedup vs baseline: 1.0079x; 1.0079x over previous
import functools

import jax
import jax.numpy as jnp
from jax import lax
from jax.experimental import pallas as pl
from jax.experimental.pallas import tpu as pltpu
from jax.experimental.pallas import tpu_sc as plsc

_NS = 16
_L = 16
_N = 65536
_NW = _NS
_CHUNK = _N // _NW
_NVEC = _CHUNK // _L
_NBIN = 16
_UNROLL = 4
_NBANK = 4


def _sc_body(idx_hbm, vals_hbm, out_hbm,
             rc_v, val_v, acc_v, part_v, red_v, out_v, shared,
             sem_a, sem_b):
    s = lax.axis_index("s")
    base = s * _CHUNK
    cp_i = pltpu.async_copy(idx_hbm.at[:, pl.ds(base, _CHUNK)], rc_v, sem_a)
    cp_v = pltpu.async_copy(vals_hbm.at[pl.ds(base, _CHUNK)], val_v, sem_b)

    zero = jnp.zeros((_L,), jnp.float32)
    for i in range(_NBANK * _NBIN):
        acc_v[pl.ds(i * _L, _L)] = zero

    lane16 = lax.iota(jnp.int32, _L) * _NBIN
    cp_i.wait()
    cp_v.wait()

    @plsc.parallel_loop(0, _NVEC, 1, unroll=_UNROLL)
    def _loop(i):
        off = i * _L
        r = rc_v[0, pl.ds(off, _L)]
        cc = rc_v[1, pl.ds(off, _L)]
        v = val_v[pl.ds(off, _L)]
        bank = (i % _NBANK) * (_NBIN * _L)
        idx = bank + lane16 + r * 4 + cc
        plsc.addupdate_scatter(acc_v, [idx], v)

    part = acc_v[pl.ds(0, _L)]
    for l in range(1, _NBANK * _NS):
        part = part + acc_v[pl.ds(l * _L, _L)]
    part_v[...] = part

    pltpu.sync_copy(part_v, shared.at[pl.ds(s * _L, _L)])
    plsc.subcore_barrier()

    @pl.when(s == 0)
    def _():
        pltpu.sync_copy(shared, red_v)
        tot = red_v[pl.ds(0, _L)]
        for l in range(1, _NS):
            tot = tot + red_v[pl.ds(l * _L, _L)]
        lane = lax.iota(jnp.int32, _L)
        plsc.store_scatter(out_v, [lane // 4, lane % 4], tot)
        pltpu.sync_copy(out_v, out_hbm)


_sc_scatter = functools.partial(
    pl.kernel,
    out_type=jax.ShapeDtypeStruct((4, 4), jnp.float32),
    mesh=plsc.VectorSubcoreMesh(
        core_axis_name="c", subcore_axis_name="s", num_cores=1),
    compiler_params=pltpu.CompilerParams(needs_layout_passes=False),
    scratch_types=[
        pltpu.VMEM((2, _CHUNK), jnp.int32),
        pltpu.VMEM((_CHUNK,), jnp.float32),
        pltpu.VMEM((_NBANK * _NBIN * _L,), jnp.float32),
        pltpu.VMEM((_L,), jnp.float32),
        pltpu.VMEM((_NS * _L,), jnp.float32),
        pltpu.VMEM((4, 4), jnp.float32),
        pltpu.VMEM_SHARED((_NS * _L,), jnp.float32),
        pltpu.SemaphoreType.DMA,
        pltpu.SemaphoreType.DMA,
    ],
)(_sc_body)


def kernel(indices, values):
    return _sc_scatter(indices.astype(jnp.int32), values)
